# Initial kernel scaffold; baseline (speedup 1.0000x reference)
import jax
import jax.numpy as jnp
from jax.experimental import pallas as pl


def kernel(x, wg, w1, w2, w3):
    raise NotImplementedError("write your pallas kernel here")


# speedup vs baseline: 2.2920x; 2.2920x over previous
import functools
import jax
import jax.numpy as jnp
from jax.experimental import pallas as pl
from jax.experimental.pallas import tpu as pltpu

E = 8
K = 2
D = 1024
FF = 4096
T = 2048
B = 256
NBLK = (T * K) // B + E
P = NBLK * B
FC = 1024
NFF = FF // FC


def _router_body(x_ref, wg_ref, logits_ref, e1_ref, e2_ref, g1_ref, g2_ref):
    x = x_ref[...]
    wg = wg_ref[...]
    logits = jax.lax.dot_general(
        x, wg, (((1,), (1,)), ((), ())), preferred_element_type=jnp.float32)
    logits_ref[...] = logits
    m = jnp.max(logits, axis=1, keepdims=True)
    ex = jnp.exp(logits - m)
    gates = ex / jnp.sum(ex, axis=1, keepdims=True)
    iota = jax.lax.broadcasted_iota(jnp.int32, (T, E), 1)
    g1 = jnp.max(gates, axis=1, keepdims=True)
    i1 = jnp.min(jnp.where(gates == g1, iota, E), axis=1, keepdims=True)
    gm = jnp.where(iota == i1, -jnp.inf, gates)
    g2 = jnp.max(gm, axis=1, keepdims=True)
    i2 = jnp.min(jnp.where(gm == g2, iota, E), axis=1, keepdims=True)
    s = g1 + g2
    e1_ref[...] = i1
    e2_ref[...] = i2
    g1_ref[...] = g1 / s
    g2_ref[...] = g2 / s


def _router(xf, wg):
    return pl.pallas_call(
        _router_body,
        out_shape=(
            jax.ShapeDtypeStruct((T, E), jnp.float32),
            jax.ShapeDtypeStruct((T, 1), jnp.int32),
            jax.ShapeDtypeStruct((T, 1), jnp.int32),
            jax.ShapeDtypeStruct((T, 1), jnp.float32),
            jax.ShapeDtypeStruct((T, 1), jnp.float32),
        ),
    )(xf, wg)


def _gemm_body(bmap_ref, xs_ref, w1_ref, w2_ref, w3_ref, ws_ref, out_ref):
    ffc = pl.program_id(1)
    x = xs_ref[...]
    a = jnp.dot(x, w1_ref[0], preferred_element_type=jnp.float32)
    b = jnp.dot(x, w2_ref[0], preferred_element_type=jnp.float32)
    h = (a / (1.0 + jnp.exp(-a))) * b
    contrib = jnp.dot(h, w3_ref[0], preferred_element_type=jnp.float32)

    @pl.when(ffc == 0)
    def _():
        out_ref[...] = jnp.zeros_like(out_ref)

    out_ref[...] += contrib

    @pl.when(ffc == NFF - 1)
    def _():
        out_ref[...] *= ws_ref[...]


def _grouped_gemm(bmap, xs, w1, w2, w3, ws):
    grid_spec = pltpu.PrefetchScalarGridSpec(
        num_scalar_prefetch=1,
        grid=(NBLK, NFF),
        in_specs=[
            pl.BlockSpec((B, D), lambda b, f, m: (b, 0)),
            pl.BlockSpec((1, D, FC), lambda b, f, m: (m[b], 0, f)),
            pl.BlockSpec((1, D, FC), lambda b, f, m: (m[b], 0, f)),
            pl.BlockSpec((1, FC, D), lambda b, f, m: (m[b], f, 0)),
            pl.BlockSpec((B, 1), lambda b, f, m: (b, 0)),
        ],
        out_specs=pl.BlockSpec((B, D), lambda b, f, m: (b, 0)),
    )
    return pl.pallas_call(
        _gemm_body,
        grid_spec=grid_spec,
        out_shape=jax.ShapeDtypeStruct((P, D), jnp.float32),
    )(bmap, xs, w1, w2, w3, ws)


def kernel(x, wg, w1, w2, w3):
    hidden_shape = x.shape
    xf = x.reshape(-1, D)

    logits, e1, e2, g1, g2 = _router(xf, wg)
    flat = jnp.concatenate([e1, e2], axis=1).reshape(-1)
    wflat = jnp.concatenate([g1, g2], axis=1).reshape(-1)

    counts = jnp.sum(flat[:, None] == jnp.arange(E)[None, :], axis=0)
    nblk = (counts + B - 1) // B
    blk_end = jnp.cumsum(nblk)
    base = (blk_end - nblk) * B
    seg_start = jnp.cumsum(counts) - counts
    order = jnp.argsort(flat, stable=True)
    e_sorted = flat[order]
    dstpos = base[e_sorted] + (jnp.arange(T * K, dtype=jnp.int32)
                               - seg_start[e_sorted])
    dst = jnp.zeros((T * K,), jnp.int32).at[order].set(dstpos)
    row_ids = jnp.zeros((P,), jnp.int32).at[dstpos].set(
        (order // K).astype(jnp.int32))
    ws = jnp.zeros((P,), jnp.float32).at[dstpos].set(wflat[order])
    bidx = jnp.arange(NBLK, dtype=jnp.int32)
    bmap = jnp.minimum(
        jnp.sum(bidx[:, None] >= blk_end[None, :], axis=1), E - 1
    ).astype(jnp.int32)

    xs = xf[row_ids]

    op = _grouped_gemm(bmap, xs, w1, w2, w3, ws.reshape(P, 1))

    dst2 = dst.reshape(T, K)
    out = op[dst2[:, 0]] + op[dst2[:, 1]]
    return out.reshape(hidden_shape), logits


# speedup vs baseline: 2.4820x; 1.0829x over previous
import functools
import jax
import jax.numpy as jnp
from jax.experimental import pallas as pl
from jax.experimental.pallas import tpu as pltpu

E = 8
K = 2
D = 1024
FF = 4096
T = 2048
B = 128
NBLK = (T * K) // B + E
P = NBLK * B
FC = 1024
NFF = FF // FC


def _router_body(x_ref, wg_ref, logits_ref, e1_ref, e2_ref, g1_ref, g2_ref):
    x = x_ref[...]
    wg = wg_ref[...]
    logits = jax.lax.dot_general(
        x, wg, (((1,), (1,)), ((), ())), preferred_element_type=jnp.float32)
    logits_ref[...] = logits
    m = jnp.max(logits, axis=1, keepdims=True)
    ex = jnp.exp(logits - m)
    gates = ex / jnp.sum(ex, axis=1, keepdims=True)
    iota = jax.lax.broadcasted_iota(jnp.int32, (T, E), 1)
    g1 = jnp.max(gates, axis=1, keepdims=True)
    i1 = jnp.min(jnp.where(gates == g1, iota, E), axis=1, keepdims=True)
    gm = jnp.where(iota == i1, -jnp.inf, gates)
    g2 = jnp.max(gm, axis=1, keepdims=True)
    i2 = jnp.min(jnp.where(gm == g2, iota, E), axis=1, keepdims=True)
    s = g1 + g2
    e1_ref[...] = i1
    e2_ref[...] = i2
    g1_ref[...] = g1 / s
    g2_ref[...] = g2 / s


def _router(xf, wg):
    return pl.pallas_call(
        _router_body,
        out_shape=(
            jax.ShapeDtypeStruct((T, E), jnp.float32),
            jax.ShapeDtypeStruct((T, 1), jnp.int32),
            jax.ShapeDtypeStruct((T, 1), jnp.int32),
            jax.ShapeDtypeStruct((T, 1), jnp.float32),
            jax.ShapeDtypeStruct((T, 1), jnp.float32),
        ),
    )(xf, wg)


def _gemm_body(bmap_ref, xs_ref, w1_ref, w2_ref, w3_ref, ws_ref, out_ref,
               acc_ref):
    f = pl.program_id(0)
    b = pl.program_id(1)
    x = xs_ref[...]
    a = jnp.dot(x, w1_ref[0], preferred_element_type=jnp.float32)
    g = jnp.dot(x, w2_ref[0], preferred_element_type=jnp.float32)
    h = (a / (1.0 + jnp.exp(-a))) * g
    contrib = jnp.dot(h, w3_ref[0], preferred_element_type=jnp.float32)

    rows = pl.ds(b * B, B)

    @pl.when(f == 0)
    def _():
        acc_ref[rows, :] = contrib

    @pl.when(jnp.logical_and(f > 0, f < NFF - 1))
    def _():
        acc_ref[rows, :] += contrib

    @pl.when(f == NFF - 1)
    def _():
        out_ref[...] = (acc_ref[rows, :] + contrib) * ws_ref[...]


def _grouped_gemm(bmap, xs, w1, w2, w3, ws):
    grid_spec = pltpu.PrefetchScalarGridSpec(
        num_scalar_prefetch=1,
        grid=(NFF, NBLK),
        in_specs=[
            pl.BlockSpec((B, D), lambda f, b, m: (b, 0)),
            pl.BlockSpec((1, D, FC), lambda f, b, m: (m[b], 0, f)),
            pl.BlockSpec((1, D, FC), lambda f, b, m: (m[b], 0, f)),
            pl.BlockSpec((1, FC, D), lambda f, b, m: (m[b], f, 0)),
            pl.BlockSpec((B, 1), lambda f, b, m: (b, 0)),
        ],
        out_specs=pl.BlockSpec((B, D), lambda f, b, m: (b, 0)),
        scratch_shapes=[pltpu.VMEM((P, D), jnp.float32)],
    )
    return pl.pallas_call(
        _gemm_body,
        grid_spec=grid_spec,
        out_shape=jax.ShapeDtypeStruct((P, D), jnp.float32),
        compiler_params=pltpu.CompilerParams(
            vmem_limit_bytes=110 * 1024 * 1024),
    )(bmap, xs, w1, w2, w3, ws)


def kernel(x, wg, w1, w2, w3):
    hidden_shape = x.shape
    xf = x.reshape(-1, D)

    logits, e1, e2, g1, g2 = _router(xf, wg)
    flat = jnp.concatenate([e1, e2], axis=1).reshape(-1)
    wflat = jnp.concatenate([g1, g2], axis=1).reshape(-1)

    counts = jnp.sum(flat[:, None] == jnp.arange(E)[None, :], axis=0)
    nblk = (counts + B - 1) // B
    blk_end = jnp.cumsum(nblk)
    base = (blk_end - nblk) * B
    seg_start = jnp.cumsum(counts) - counts
    order = jnp.argsort(flat, stable=True)
    e_sorted = flat[order]
    dstpos = base[e_sorted] + (jnp.arange(T * K, dtype=jnp.int32)
                               - seg_start[e_sorted])
    dst = jnp.zeros((T * K,), jnp.int32).at[order].set(dstpos)
    row_ids = jnp.zeros((P,), jnp.int32).at[dstpos].set(
        (order // K).astype(jnp.int32))
    ws = jnp.zeros((P,), jnp.float32).at[dstpos].set(wflat[order])
    bidx = jnp.arange(NBLK, dtype=jnp.int32)
    bmap = jnp.minimum(
        jnp.sum(bidx[:, None] >= blk_end[None, :], axis=1), E - 1
    ).astype(jnp.int32)

    xs = xf[row_ids]

    op = _grouped_gemm(bmap, xs, w1, w2, w3, ws.reshape(P, 1))

    dst2 = dst.reshape(T, K)
    out = op[dst2[:, 0]] + op[dst2[:, 1]]
    return out.reshape(hidden_shape), logits


# speedup vs baseline: 2.5838x; 1.0410x over previous
import functools
import jax
import jax.numpy as jnp
from jax import lax
from jax.experimental import pallas as pl
from jax.experimental.pallas import tpu as pltpu
from jax.experimental.pallas import tpu_sc as plsc

E = 8
K = 2
D = 1024
FF = 4096
T = 2048
B = 128
NBLK = (T * K) // B + E
P = NBLK * B
FC = 1024
NFF = FF // FC

BMAP_PAD = 64
NC, NS = 2, 16
NW = NC * NS
RPW = P // NW
GCH = 80
TPW = T // NW
CCH = 32

_mesh = plsc.VectorSubcoreMesh(core_axis_name="c", subcore_axis_name="s")


def _router_body(x_ref, wg_ref, logits_ref, e1_ref, e2_ref, g1_ref, g2_ref):
    x = x_ref[...]
    wg = wg_ref[...]
    logits = jax.lax.dot_general(
        x, wg, (((1,), (1,)), ((), ())), preferred_element_type=jnp.float32)
    logits_ref[...] = logits
    m = jnp.max(logits, axis=1, keepdims=True)
    ex = jnp.exp(logits - m)
    gates = ex / jnp.sum(ex, axis=1, keepdims=True)
    iota = jax.lax.broadcasted_iota(jnp.int32, (T, E), 1)
    g1 = jnp.max(gates, axis=1, keepdims=True)
    i1 = jnp.min(jnp.where(gates == g1, iota, E), axis=1, keepdims=True)
    gm = jnp.where(iota == i1, -jnp.inf, gates)
    g2 = jnp.max(gm, axis=1, keepdims=True)
    i2 = jnp.min(jnp.where(gm == g2, iota, E), axis=1, keepdims=True)
    s = g1 + g2
    e1_ref[...] = i1
    e2_ref[...] = i2
    g1_ref[...] = g1 / s
    g2_ref[...] = g2 / s


def _router(xf, wg):
    return pl.pallas_call(
        _router_body,
        out_shape=(
            jax.ShapeDtypeStruct((T, E), jnp.float32),
            jax.ShapeDtypeStruct((T, 1), jnp.int32),
            jax.ShapeDtypeStruct((T, 1), jnp.int32),
            jax.ShapeDtypeStruct((T, 1), jnp.float32),
            jax.ShapeDtypeStruct((T, 1), jnp.float32),
        ),
    )(xf, wg)


def _gemm_body(bmap_ref, xs_ref, w1_ref, w2_ref, w3_ref, ws_ref, out_ref,
               acc_ref):
    f = pl.program_id(0)
    b = pl.program_id(1)
    x = xs_ref[...]
    a = jnp.dot(x, w1_ref[0], preferred_element_type=jnp.float32)
    g = jnp.dot(x, w2_ref[0], preferred_element_type=jnp.float32)
    h = (a / (1.0 + jnp.exp(-a))) * g
    contrib = jnp.dot(h, w3_ref[0], preferred_element_type=jnp.float32)

    rows = pl.ds(b * B, B)

    @pl.when(f == 0)
    def _():
        acc_ref[rows, :] = contrib

    @pl.when(jnp.logical_and(f > 0, f < NFF - 1))
    def _():
        acc_ref[rows, :] += contrib

    @pl.when(f == NFF - 1)
    def _():
        out_ref[...] = (acc_ref[rows, :] + contrib) * ws_ref[...]


def _grouped_gemm(bmap, xs, w1, w2, w3, ws):
    grid_spec = pltpu.PrefetchScalarGridSpec(
        num_scalar_prefetch=1,
        grid=(NFF, NBLK),
        in_specs=[
            pl.BlockSpec((B, D), lambda f, b, m: (b, 0)),
            pl.BlockSpec((1, D, FC), lambda f, b, m: (m[b], 0, f)),
            pl.BlockSpec((1, D, FC), lambda f, b, m: (m[b], 0, f)),
            pl.BlockSpec((1, FC, D), lambda f, b, m: (m[b], f, 0)),
            pl.BlockSpec((B, 1), lambda f, b, m: (b, 0)),
        ],
        out_specs=pl.BlockSpec((B, D), lambda f, b, m: (b, 0)),
        scratch_shapes=[pltpu.VMEM((P, D), jnp.float32)],
    )
    return pl.pallas_call(
        _gemm_body,
        grid_spec=grid_spec,
        out_shape=jax.ShapeDtypeStruct((P, D), jnp.float32),
        compiler_params=pltpu.CompilerParams(
            vmem_limit_bytes=110 * 1024 * 1024),
    )(bmap, xs, w1, w2, w3, ws)


NV = T // 16


def _sort_body(e1_hbm, e2_hbm, g1_hbm, g2_hbm,
               dst_hbm, row_hbm, ws_hbm, bmap_hbm,
               e1v, e2v, g1v, g2v, dstv, rowv, wsv, mapv):
    cid = lax.axis_index("c")
    sid = lax.axis_index("s")

    @pl.when(jnp.logical_and(cid == 0, sid == 0))
    def _():
        pltpu.sync_copy(e1_hbm, e1v)
        pltpu.sync_copy(e2_hbm, e2v)
        pltpu.sync_copy(g1_hbm, g1v)
        pltpu.sync_copy(g2_hbm, g2v)

        iot = lax.iota(jnp.int32, 16)
        zi = jnp.zeros((16,), jnp.int32)
        zf = jnp.zeros((16,), jnp.float32)

        def zrow(i, _):
            rowv[pl.ds(i * 16, 16)] = zi
            wsv[pl.ds(i * 16, 16)] = zf
            return 0
        lax.fori_loop(0, P // 16, zrow, 0)

        def hbody(i, cnts):
            va = e1v[pl.ds(16 * i, 16)]
            vb = e2v[pl.ds(16 * i, 16)]
            out = []
            for e in range(E):
                ca = jnp.sum(jnp.where(va == e, 1, 0))
                cb = jnp.sum(jnp.where(vb == e, 1, 0))
                out.append(cnts[e] + ca + cb)
            return tuple(out)
        zero = jnp.zeros((), jnp.int32)
        cnts = lax.fori_loop(0, NV, hbody, (zero,) * E)

        cnt_vec = zi
        for e in range(E):
            cnt_vec = jnp.where(iot == e, cnts[e], cnt_vec)
        nblk_vec = (cnt_vec + B - 1) // B
        end_vec = jnp.cumsum(nblk_vec)
        base_vec = (end_vec - nblk_vec) * B
        used = end_vec[E - 1]
        nz = jnp.where(jnp.logical_and(nblk_vec > 0, iot < E), iot, 0)
        last_e = jnp.max(nz)
        ends = [end_vec[e] for e in range(E)]
        for j in range(BMAP_PAD // 16):
            bidx = iot + 16 * j
            val = zi
            for e in range(E):
                val = val + jnp.where(bidx >= ends[e], 1, 0)
            mval = jnp.where(bidx < used, jnp.minimum(val, E - 1), last_e)
            mapv[pl.ds(16 * j, 16)] = mval

        def make_abody(ev, gv_ref, off):
            def abody(i, runs):
                va = ev[pl.ds(16 * i, 16)]
                gv = gv_ref[pl.ds(16 * i, 16)]
                tok = iot + 16 * i
                slot = zi
                out = []
                for e in range(E):
                    ind = jnp.where(va == e, 1, 0)
                    c = jnp.cumsum(ind)
                    slot = jnp.where(va == e, runs[e] + c - 1, slot)
                    out.append(runs[e] + c[15])
                plsc.store_scatter(rowv, [slot], tok)
                plsc.store_scatter(wsv, [slot], gv)
                dstv[pl.ds(off + 16 * i, 16)] = slot
                return tuple(out)
            return abody

        runs = tuple(base_vec[e] for e in range(E))
        runs = lax.fori_loop(0, NV, make_abody(e1v, g1v, 0), runs)
        lax.fori_loop(0, NV, make_abody(e2v, g2v, T), runs)

        pltpu.sync_copy(dstv, dst_hbm)
        pltpu.sync_copy(rowv, row_hbm)
        pltpu.sync_copy(wsv, ws_hbm)
        pltpu.sync_copy(mapv, bmap_hbm)


def _sc_sort(e1, e2, g1, g2):
    f = pl.kernel(
        _sort_body,
        out_type=(
            jax.ShapeDtypeStruct((K * T,), jnp.int32),
            jax.ShapeDtypeStruct((P,), jnp.int32),
            jax.ShapeDtypeStruct((P,), jnp.float32),
            jax.ShapeDtypeStruct((BMAP_PAD,), jnp.int32),
        ),
        mesh=_mesh,
        scratch_types=[
            pltpu.VMEM((T,), jnp.int32),
            pltpu.VMEM((T,), jnp.int32),
            pltpu.VMEM((T,), jnp.float32),
            pltpu.VMEM((T,), jnp.float32),
            pltpu.VMEM((K * T,), jnp.int32),
            pltpu.VMEM((P,), jnp.int32),
            pltpu.VMEM((P,), jnp.float32),
            pltpu.VMEM((BMAP_PAD,), jnp.int32),
        ],
        compiler_params=pltpu.CompilerParams(needs_layout_passes=False),
    )
    return f(e1, e2, g1, g2)


def _gather_body(x_hbm, row_hbm, xs_hbm, idxv, bufv, sem):
    cid = lax.axis_index("c")
    sid = lax.axis_index("s")
    wid = sid * NC + cid
    base = wid * RPW

    def cbody(ci, _):
        pltpu.sync_copy(row_hbm.at[pl.ds(base + ci * GCH, GCH)], idxv)
        pltpu.async_copy(x_hbm.at[idxv], bufv, sem).wait()
        pltpu.sync_copy(bufv, xs_hbm.at[pl.ds(base + ci * GCH, GCH)])
        return 0
    lax.fori_loop(0, RPW // GCH, cbody, 0)


def _sc_gather(xf, row_ids):
    f = pl.kernel(
        _gather_body,
        out_type=jax.ShapeDtypeStruct((P, D), jnp.float32),
        mesh=_mesh,
        scratch_types=[
            pltpu.VMEM((GCH,), jnp.int32),
            pltpu.VMEM((GCH, D), jnp.float32),
            pltpu.SemaphoreType.DMA,
        ],
    )
    return f(xf, row_ids)


def _combine_body(op_hbm, dst_hbm, out_hbm, idxv, gbufv, obufv, sem):
    cid = lax.axis_index("c")
    sid = lax.axis_index("s")
    wid = sid * NC + cid
    tbase = wid * TPW

    def cbody(ci, _):
        tb = tbase + ci * CCH
        pltpu.sync_copy(dst_hbm.at[pl.ds(tb, CCH)], idxv.at[pl.ds(0, CCH)])
        pltpu.sync_copy(dst_hbm.at[pl.ds(T + tb, CCH)],
                        idxv.at[pl.ds(CCH, CCH)])
        pltpu.async_copy(op_hbm.at[idxv], gbufv, sem).wait()

        def jbody(j, _):
            def qbody(q, _):
                s = pl.ds(q * 16, 16)
                obufv[j, s] = gbufv[j, s] + gbufv[CCH + j, s]
                return 0
            lax.fori_loop(0, D // 16, qbody, 0)
            return 0
        lax.fori_loop(0, CCH, jbody, 0)
        pltpu.sync_copy(obufv, out_hbm.at[pl.ds(tb, CCH)])
        return 0
    lax.fori_loop(0, TPW // CCH, cbody, 0)


def _sc_combine(op, dst):
    f = pl.kernel(
        _combine_body,
        out_type=jax.ShapeDtypeStruct((T, D), jnp.float32),
        mesh=_mesh,
        scratch_types=[
            pltpu.VMEM((2 * CCH,), jnp.int32),
            pltpu.VMEM((2 * CCH, D), jnp.float32),
            pltpu.VMEM((CCH, D), jnp.float32),
            pltpu.SemaphoreType.DMA,
        ],
    )
    return f(op, dst)


def kernel(x, wg, w1, w2, w3):
    hidden_shape = x.shape
    xf = x.reshape(-1, D)

    logits, e1, e2, g1, g2 = _router(xf, wg)
    dst, row_ids, ws, bmap = _sc_sort(
        e1.reshape(T), e2.reshape(T), g1.reshape(T), g2.reshape(T))
    xs = _sc_gather(xf, row_ids)
    op = _grouped_gemm(bmap, xs, w1, w2, w3, ws.reshape(P, 1))
    out = _sc_combine(op, dst)
    return out.reshape(hidden_shape), logits
